# baseline (speedup 1.0000x reference)
import jax
import jax.numpy as jnp
from jax import lax
from jax.experimental import pallas as pl
from jax.experimental.pallas import tpu as pltpu
from jax.experimental.pallas import tpu_sc as plsc

SEQ = 4096
EMB = 128
BATCH = 4
NUM_INPUTS = 2
NUM_CORES = 2
NUM_SUBCORES = 16
NW = NUM_CORES * NUM_SUBCORES
WROWS = 2 * SEQ // NW
CH = 32
NCHUNK = WROWS // CH
XROWS = CH + 8
PROWS = WROWS + 8
LANES = 16
GROUPS = EMB // LANES


def _body(x0, x1, upw, lpw, rtw, out,
          pb, xb0, xb1, ob0, ob1, rb, lp_v, rot_v,
          s_in0, s_in1, s_out0, s_out1, s_small):
    cid = lax.axis_index("c")
    sid = lax.axis_index("s")
    w = sid * NUM_CORES + cid
    a0 = pl.multiple_of(w * WROWS, WROWS)

    d_lp = pltpu.async_copy(lpw, lp_v, s_small)
    d_rt = pltpu.async_copy(rtw, rot_v, s_small)

    @pl.when(w <= 15)
    def _():
        base = pl.multiple_of(jnp.maximum(a0 - 8, 0), 8)
        pltpu.async_copy(upw.at[pl.ds(base, PROWS)], pb, s_small)

    @pl.when(w == 16)
    def _():
        pltpu.async_copy(upw.at[pl.ds(SEQ - 8, 8)], pb.at[pl.ds(0, 8)], s_small)
        pltpu.async_copy(upw.at[pl.ds(0, PROWS - 8)], pb.at[pl.ds(8, PROWS - 8)], s_small)

    @pl.when(w >= 17)
    def _():
        base = pl.multiple_of(a0 - SEQ - 8, 8)
        pltpu.async_copy(upw.at[pl.ds(base, PROWS)], pb, s_small)

    pshift = jnp.where(w == 0, -1, 7)

    xbufs = (xb0, xb1)
    obufs = (ob0, ob1)
    in_sems = (s_in0, s_in1)
    out_sems = (s_out0, s_out1)

    def start_in(h):
        xb, sem = xbufs[h % 2], in_sems[h % 2]
        ah = pl.multiple_of(a0 + h * CH, CH)
        if h == 0:
            @pl.when(w <= 15)
            def _():
                base = pl.multiple_of(jnp.maximum(ah - 8, 0), 8)
                pltpu.async_copy(x0.at[:, pl.ds(base, XROWS)], xb, sem)

            @pl.when(w == 16)
            def _():
                pltpu.async_copy(x0.at[:, pl.ds(SEQ - 8, 8)], xb.at[:, pl.ds(0, 8)], sem)
                pltpu.async_copy(x1.at[:, pl.ds(0, CH)], xb.at[:, pl.ds(8, CH)], sem)

            @pl.when(w >= 17)
            def _():
                base = pl.multiple_of(ah - SEQ - 8, 8)
                pltpu.async_copy(x1.at[:, pl.ds(base, XROWS)], xb, sem)
        else:
            @pl.when(w <= 15)
            def _():
                base = pl.multiple_of(ah - 8, 8)
                pltpu.async_copy(x0.at[:, pl.ds(base, XROWS)], xb, sem)

            @pl.when(w >= 16)
            def _():
                base = pl.multiple_of(ah - SEQ - 8, 8)
                pltpu.async_copy(x1.at[:, pl.ds(base, XROWS)], xb, sem)

    def wait_in(h):
        pltpu.make_async_copy(x0.at[:, pl.ds(0, XROWS)],
                              xbufs[h % 2], in_sems[h % 2]).wait()

    start_in(0)

    d_lp.wait()
    d_rt.wait()
    pltpu.make_async_copy(upw.at[pl.ds(0, PROWS)], pb, s_small).wait()

    lp0 = [lp_v[0, pl.ds(g * LANES, LANES)] for g in range(GROUPS)]
    lp1 = [lp_v[1, pl.ds(g * LANES, LANES)] for g in range(GROUPS)]
    @pl.when(w == 16)
    def _():
        for r in range(8):
            for g in range(GROUPS):
                col = pl.ds(g * LANES, LANES)
                pb[r, col] = pb[r, col] + (lp0[g] - lp1[g])

    m_lo = jnp.where(w <= 15, jnp.float32(1.0), jnp.float32(0.0))
    lp_sel = [lp1[g] + (lp0[g] - lp1[g]) * m_lo for g in range(GROUPS)]

    def start_in_dyn(k, xb, sem):
        ah = pl.multiple_of(a0 + k * CH, 8)

        @pl.when(w <= 15)
        def _():
            pltpu.async_copy(x0.at[:, pl.ds(ah - 8, XROWS)], xb, sem)

        @pl.when(w >= 16)
        def _():
            pltpu.async_copy(x1.at[:, pl.ds(ah - SEQ - 8, XROWS)], xb, sem)

    def wait_in_buf(xb, sem):
        pltpu.make_async_copy(x0.at[:, pl.ds(0, XROWS)], xb, sem).wait()

    def wait_out_buf(ob, sem):
        pltpu.make_async_copy(ob, out.at[pl.ds(0, CH)], sem).wait()

    def compute(k, xshift, xb, ob):
        poff = k * CH + pshift

        @plsc.parallel_loop(0, CH, unroll=2)
        def _(r):
            xi = jnp.maximum(r + xshift, 0)
            pi = jnp.maximum(r + poff, 0)
            for g in range(GROUPS):
                col = pl.ds(g * LANES, LANES)
                pv = pb[pi, col] + lp_sel[g]
                for b in range(BATCH):
                    ob[r, b, col] = xb[b, xi, col] + pv

    def start_out(k, ob, sem):
        ah = pl.multiple_of(a0 + k * CH, 8)
        pltpu.async_copy(ob, out.at[pl.ds(ah, CH)], sem)

    start_in(1)
    wait_in_buf(xb0, s_in0)
    compute(0, pshift, xb0, ob0)

    @pl.when(w == 0)
    def _():
        for b in range(BATCH):
            for g in range(GROUPS):
                col = pl.ds(g * LANES, LANES)
                ob0[0, b, col] = rot_v[0, col]

    start_out(0, ob0, s_out0)

    start_in_dyn(jnp.int32(2), xb0, s_in0)
    wait_in_buf(xb1, s_in1)
    compute(1, 7, xb1, ob1)
    start_out(1, ob1, s_out1)

    def round_body(t, acc):
        k1 = 2 * t + 2

        start_in_dyn(k1 + 1, xb1, s_in1)
        wait_in_buf(xb0, s_in0)
        wait_out_buf(ob0, s_out0)
        compute(k1, 7, xb0, ob0)
        start_out(k1, ob0, s_out0)

        @pl.when(k1 + 2 < NCHUNK)
        def _():
            start_in_dyn(k1 + 2, xb0, s_in0)

        wait_in_buf(xb1, s_in1)
        wait_out_buf(ob1, s_out1)
        compute(k1 + 1, 7, xb1, ob1)
        start_out(k1 + 1, ob1, s_out1)
        return acc

    lax.fori_loop(0, NCHUNK // 2 - 1, round_body, 0)

    wait_out_buf(ob0, s_out0)
    wait_out_buf(ob1, s_out1)

    @pl.when(w == NW - 1)
    def _():
        for g in range(GROUPS):
            col = pl.ds(g * LANES, LANES)
            pv = pb[PROWS - 1, col] + lp_sel[g]
            for b in range(BATCH):
                rb[0, b, col] = xb1[b, XROWS - 1, col] + pv
        pltpu.async_copy(rb, out.at[pl.ds(NUM_INPUTS * SEQ, 1)], s_small)
        pltpu.make_async_copy(rb, out.at[pl.ds(NUM_INPUTS * SEQ, 1)],
                              s_small).wait()


def kernel(x0, x1, unique_pos_w, layer_pos_w, rot_token_w):
    mesh = plsc.VectorSubcoreMesh(core_axis_name="c", subcore_axis_name="s")
    f32 = jnp.float32
    run = pl.kernel(
        _body,
        out_type=jax.ShapeDtypeStruct((NUM_INPUTS * SEQ + 1, BATCH, EMB), f32),
        mesh=mesh,
        scratch_types=[
            pltpu.VMEM((PROWS, EMB), f32),
            pltpu.VMEM((BATCH, XROWS, EMB), f32),
            pltpu.VMEM((BATCH, XROWS, EMB), f32),
            pltpu.VMEM((CH, BATCH, EMB), f32),
            pltpu.VMEM((CH, BATCH, EMB), f32),
            pltpu.VMEM((1, BATCH, EMB), f32),
            pltpu.VMEM((NUM_INPUTS, EMB), f32),
            pltpu.VMEM((1, EMB), f32),
            pltpu.SemaphoreType.DMA,
            pltpu.SemaphoreType.DMA,
            pltpu.SemaphoreType.DMA,
            pltpu.SemaphoreType.DMA,
            pltpu.SemaphoreType.DMA,
        ],
    )
    p = run(x0, x1, unique_pos_w, layer_pos_w, rot_token_w)
    return jnp.transpose(p, (1, 0, 2))


# speedup vs baseline: 1.0783x; 1.0783x over previous
import jax
import jax.numpy as jnp
from jax import lax
from jax.experimental import pallas as pl
from jax.experimental.pallas import tpu as pltpu
from jax.experimental.pallas import tpu_sc as plsc

SEQ = 4096
EMB = 128
BATCH = 4
NUM_INPUTS = 2
NUM_CORES = 2
NUM_SUBCORES = 16
NW = NUM_CORES * NUM_SUBCORES
WROWS = SEQ // NW
CH = 32
NCHUNK = NUM_INPUTS * WROWS // CH
LANES = 16
GROUPS = EMB // LANES


def _body(x0, x1, upw, lpw, rtw, out,
          pb, xb0, xb1, ob0, ob1, rv, lp_v, rot_v,
          s_in0, s_in1, s_out0, s_out1, s_small):
    cid = lax.axis_index("c")
    sid = lax.axis_index("s")
    w = sid * NUM_CORES + cid
    s0 = pl.multiple_of(w * WROWS, WROWS)

    d_lp = pltpu.async_copy(lpw, lp_v, s_small)
    d_rt = pltpu.async_copy(rtw, rot_v, s_small)
    d_pb = pltpu.async_copy(upw.at[pl.ds(s0, WROWS)], pb, s_small)

    def start_in(k, xb, sem):
        @pl.when(k <= 3)
        def _():
            ib = pl.multiple_of(s0 + k * CH, CH)
            pltpu.async_copy(x0.at[:, pl.ds(ib, CH)], xb, sem)

        @pl.when(k >= 4)
        def _():
            ib = pl.multiple_of(s0 + (k - 4) * CH, CH)
            pltpu.async_copy(x1.at[:, pl.ds(ib, CH)], xb, sem)

    def wait_in(xb, sem):
        pltpu.make_async_copy(x0.at[:, pl.ds(0, CH)], xb, sem).wait()

    def wait_out(ob, sem):
        pltpu.make_async_copy(ob, out.at[pl.ds(0, CH)], sem).wait()

    start_in(jnp.int32(0), xb0, s_in0)
    start_in(jnp.int32(1), xb1, s_in1)

    d_lp.wait()
    d_rt.wait()
    d_pb.wait()

    lp0 = [lp_v[0, pl.ds(g * LANES, LANES)] for g in range(GROUPS)]
    lp1 = [lp_v[1, pl.ds(g * LANES, LANES)] for g in range(GROUPS)]

    def compute(k, xb, ob):
        m = jnp.where(k <= 3, jnp.float32(1.0), jnp.float32(0.0))
        lpk = [lp1[g] + (lp0[g] - lp1[g]) * m for g in range(GROUPS)]
        poff = (k - 4 * jnp.where(k >= 4, 1, 0)) * CH

        @plsc.parallel_loop(0, CH, unroll=1)
        def _(r):
            pi = poff + r
            for g in range(GROUPS):
                col = pl.ds(g * LANES, LANES)
                pv = pb[pi, col] + lpk[g]
                for b in range(BATCH):
                    ob[r, b, col] = xb[b, r, col] + pv

    def start_out(k, ob, sem):
        obase = 1 + s0 + k * CH + jnp.where(k >= 4, SEQ - 4 * CH, 0)
        pltpu.async_copy(ob, out.at[pl.ds(obase, CH)], sem)

    @pl.when(w == 0)
    def _():
        for b in range(BATCH):
            for g in range(GROUPS):
                col = pl.ds(g * LANES, LANES)
                rv[0, b, col] = rot_v[0, col]
        pltpu.async_copy(rv, out.at[pl.ds(0, 1)], s_small)
        pltpu.make_async_copy(rv, out.at[pl.ds(0, 1)], s_small).wait()

    wait_in(xb0, s_in0)
    compute(jnp.int32(0), xb0, ob0)
    start_out(jnp.int32(0), ob0, s_out0)

    start_in(jnp.int32(2), xb0, s_in0)
    wait_in(xb1, s_in1)
    compute(jnp.int32(1), xb1, ob1)
    start_out(jnp.int32(1), ob1, s_out1)

    def round_body(t, acc):
        k1 = 2 * t + 2

        start_in(k1 + 1, xb1, s_in1)
        wait_in(xb0, s_in0)
        wait_out(ob0, s_out0)
        compute(k1, xb0, ob0)
        start_out(k1, ob0, s_out0)

        @pl.when(k1 + 2 < NCHUNK)
        def _():
            start_in(k1 + 2, xb0, s_in0)

        wait_in(xb1, s_in1)
        wait_out(ob1, s_out1)
        compute(k1 + 1, xb1, ob1)
        start_out(k1 + 1, ob1, s_out1)
        return acc

    lax.fori_loop(0, NCHUNK // 2 - 1, round_body, 0)

    wait_out(ob0, s_out0)
    wait_out(ob1, s_out1)


def kernel(x0, x1, unique_pos_w, layer_pos_w, rot_token_w):
    mesh = plsc.VectorSubcoreMesh(core_axis_name="c", subcore_axis_name="s")
    f32 = jnp.float32
    run = pl.kernel(
        _body,
        out_type=jax.ShapeDtypeStruct((NUM_INPUTS * SEQ + 1, BATCH, EMB), f32),
        mesh=mesh,
        scratch_types=[
            pltpu.VMEM((WROWS, EMB), f32),
            pltpu.VMEM((BATCH, CH, EMB), f32),
            pltpu.VMEM((BATCH, CH, EMB), f32),
            pltpu.VMEM((CH, BATCH, EMB), f32),
            pltpu.VMEM((CH, BATCH, EMB), f32),
            pltpu.VMEM((1, BATCH, EMB), f32),
            pltpu.VMEM((NUM_INPUTS, EMB), f32),
            pltpu.VMEM((1, EMB), f32),
            pltpu.SemaphoreType.DMA,
            pltpu.SemaphoreType.DMA,
            pltpu.SemaphoreType.DMA,
            pltpu.SemaphoreType.DMA,
            pltpu.SemaphoreType.DMA,
        ],
    )
    p = run(x0, x1, unique_pos_w, layer_pos_w, rot_token_w)
    return jnp.transpose(p, (1, 0, 2))
